# baseline (speedup 1.0000x reference)
import jax
import jax.numpy as jnp
from jax.experimental import pallas as pl
from jax.experimental.pallas import tpu as pltpu

B, M, H, D, K = 8, 512, 8, 32, 16


def _update_body(sel_ref, probs_ref, mat_ref, norm_ref, mu_ref, nu_ref,
                 main_col_ref, main_row_ref, aux_row_ref,
                 out_mat_ref, out_norm_ref):
    b = pl.program_id(0)
    h = pl.program_id(1)
    k = pl.program_id(2)
    p = probs_ref[b, h, k]

    main_col = main_col_ref[0]
    main_row = main_row_ref[0]
    aux_row = aux_row_ref[0]

    mat_decay = jax.nn.sigmoid(main_col + aux_row)
    norm_decay = jax.nn.sigmoid(main_row)

    sel_m = mat_ref[0]
    mu = mu_ref[0]
    sel_n = norm_ref[0]
    nu = nu_ref[0]

    out_mat_ref[0] = sel_m + (p * mat_decay) * (mu - sel_m)
    out_norm_ref[0] = sel_n + (p * norm_decay) * (nu - sel_n)


def kernel(matrix, normalizer, matrix_update, normalizer_update,
           main_decay_logits, aux_decay_logits, sel_index, sel_probs):
    mat_flat = matrix.reshape(B * M * H, D, D)
    norm_flat = normalizer.reshape(B * M * H, 1, D)
    mu_flat = matrix_update.reshape(B * K * H, D, D)
    nu_flat = normalizer_update.reshape(B * K * H, 1, D)
    main_col = main_decay_logits.reshape(M * H, D, 1)
    main_row = main_decay_logits.reshape(M * H, 1, D)
    aux_row = aux_decay_logits.reshape(M, 1, D)

    def mem_row(b, h, k, sel_ref, probs_ref):
        return ((b * M + sel_ref[b, h, k]) * H + h, 0, 0)

    def upd_row(b, h, k, sel_ref, probs_ref):
        return ((b * K + k) * H + h, 0, 0)

    def main_rowmap(b, h, k, sel_ref, probs_ref):
        return (sel_ref[b, h, k] * H + h, 0, 0)

    def aux_rowmap(b, h, k, sel_ref, probs_ref):
        return (sel_ref[b, h, k], 0, 0)

    grid_spec = pltpu.PrefetchScalarGridSpec(
        num_scalar_prefetch=2,
        grid=(B, H, K),
        in_specs=[
            pl.BlockSpec((1, D, D), mem_row),
            pl.BlockSpec((1, 1, D), mem_row),
            pl.BlockSpec((1, D, D), upd_row),
            pl.BlockSpec((1, 1, D), upd_row),
            pl.BlockSpec((1, D, 1), main_rowmap),
            pl.BlockSpec((1, 1, D), main_rowmap),
            pl.BlockSpec((1, 1, D), aux_rowmap),
        ],
        out_specs=[
            pl.BlockSpec((1, D, D), mem_row),
            pl.BlockSpec((1, 1, D), mem_row),
        ],
    )

    out_mat, out_norm = pl.pallas_call(
        _update_body,
        grid_spec=grid_spec,
        out_shape=[
            jax.ShapeDtypeStruct(mat_flat.shape, mat_flat.dtype),
            jax.ShapeDtypeStruct(norm_flat.shape, norm_flat.dtype),
        ],
        input_output_aliases={2: 0, 3: 1},
    )(sel_index, sel_probs, mat_flat, norm_flat, mu_flat, nu_flat,
      main_col, main_row, aux_row)

    return (out_mat.reshape(B, M, H, D, D), out_norm.reshape(B, M, H, D))
